# baseline (speedup 1.0000x reference)
import functools

import jax
import jax.numpy as jnp
from jax import lax
from jax.experimental import pallas as pl
from jax.experimental.pallas import tpu as pltpu
from jax.experimental.pallas import tpu_sc as plsc

NUM_TABLES = 4
VOCAB = 100000
EMB = 32
B = 4096
L = 50

BLK = 128


def _sc_gather_concat(idxt, tbs):
    info = plsc.get_sparse_core_info()
    nc, ns = info.num_cores, info.num_subcores
    nw = nc * ns
    assert B == nw * BLK

    mesh = plsc.VectorSubcoreMesh(core_axis_name="c", subcore_axis_name="s")

    @functools.partial(
        pl.kernel,
        mesh=mesh,
        compiler_params=pltpu.CompilerParams(use_tc_tiling_on_sc=False),
        out_type=jax.ShapeDtypeStruct((L, B, NUM_TABLES * EMB), jnp.float32),
        scratch_types=[
            pltpu.VMEM((L, NUM_TABLES, BLK), jnp.int32),
            pltpu.VMEM((2, NUM_TABLES, BLK, EMB), jnp.float32),
            pltpu.SemaphoreType.DMA((2,)),
            pltpu.SemaphoreType.DMA((2,)),
        ],
    )
    def k(idx_hbm, tb_hbm, out_hbm, idx_v, tmp_v, gsem, wsem):
        tables = tuple(tb_hbm.at[t] for t in range(NUM_TABLES))
        wid = lax.axis_index("s") * nc + lax.axis_index("c")
        b0 = wid * BLK

        pltpu.sync_copy(idx_hbm.at[:, wid], idx_v)

        def gdescs(l, p):
            return [pltpu.make_async_copy(
                tables[t].at[idx_v.at[l, t]],
                tmp_v.at[p, t],
                gsem.at[p]) for t in range(NUM_TABLES)]

        def wdescs(l, p):
            return [pltpu.make_async_copy(
                tmp_v.at[p, t],
                out_hbm.at[l, pl.ds(b0, BLK), pl.ds(t * EMB, EMB)],
                wsem.at[p]) for t in range(NUM_TABLES)]

        for d in gdescs(0, 0):
            d.start()

        def body(l, _):
            p = lax.rem(l, 2)

            @pl.when(l + 1 < L)
            def _():
                @pl.when(l + 1 >= 2)
                def _():
                    for d in wdescs(l - 1, 1 - p):
                        d.wait()

                for d in gdescs(l + 1, 1 - p):
                    d.start()

            for d in gdescs(l, p):
                d.wait()

            for d in wdescs(l, p):
                d.start()
            return ()

        lax.fori_loop(0, L, body, (), unroll=False)
        for d in wdescs(L - 2, 0):
            d.wait()
        for d in wdescs(L - 1, 1):
            d.wait()

    return k(idxt, tbs)


def kernel(inputs, table0, table1, table2, table3):
    idxt = (inputs.astype(jnp.int32).transpose(2, 1, 0)
            .reshape(L, B // BLK, BLK, NUM_TABLES)
            .transpose(0, 1, 3, 2))
    tbs = jnp.stack([table0, table1, table2, table3])
    out = _sc_gather_concat(idxt, tbs)
    return out.transpose(1, 0, 2)


# speedup vs baseline: 1.1631x; 1.1631x over previous
import functools

import jax
import jax.numpy as jnp
from jax import lax
from jax.experimental import pallas as pl
from jax.experimental.pallas import tpu as pltpu
from jax.experimental.pallas import tpu_sc as plsc

NUM_TABLES = 4
VOCAB = 100000
EMB = 32
B = 4096
L = 50

BLK = 128
NBUF = 4
GAHEAD = 2


def _sc_gather_concat(idxt, t0, t1, t2, t3):
    info = plsc.get_sparse_core_info()
    nc, ns = info.num_cores, info.num_subcores
    nw = nc * ns
    assert B == nw * BLK

    mesh = plsc.VectorSubcoreMesh(core_axis_name="c", subcore_axis_name="s")

    @functools.partial(
        pl.kernel,
        mesh=mesh,
        compiler_params=pltpu.CompilerParams(use_tc_tiling_on_sc=False),
        out_type=jax.ShapeDtypeStruct((L, B, NUM_TABLES * EMB), jnp.float32),
        scratch_types=[
            pltpu.VMEM((L, NUM_TABLES, BLK), jnp.int32),
            pltpu.VMEM((NBUF, NUM_TABLES, BLK, EMB), jnp.float32),
            pltpu.SemaphoreType.DMA((NBUF,)),
            pltpu.SemaphoreType.DMA((NBUF,)),
        ],
    )
    def k(idx_hbm, tb0, tb1, tb2, tb3, out_hbm, idx_v, tmp_v, gsem, wsem):
        tables = (tb0, tb1, tb2, tb3)
        wid = lax.axis_index("s") * nc + lax.axis_index("c")
        b0 = wid * BLK

        pltpu.sync_copy(idx_hbm.at[:, wid], idx_v)

        def gdescs(l, p):
            return [pltpu.make_async_copy(
                tables[t].at[idx_v.at[l, t]],
                tmp_v.at[p, t],
                gsem.at[p]) for t in range(NUM_TABLES)]

        def wdescs(l, p):
            return [pltpu.make_async_copy(
                tmp_v.at[p, t],
                out_hbm.at[l, pl.ds(b0, BLK), pl.ds(t * EMB, EMB)],
                wsem.at[p]) for t in range(NUM_TABLES)]

        for l in range(GAHEAD):
            for d in gdescs(l, l):
                d.start()

        def body(l, _):
            p = lax.rem(l, NBUF)

            @pl.when(l + GAHEAD < L)
            def _():
                q = lax.rem(l + GAHEAD, NBUF)

                @pl.when(l - (NBUF - GAHEAD) >= 0)
                def _():
                    for d in wdescs(l - (NBUF - GAHEAD), q):
                        d.wait()

                for d in gdescs(l + GAHEAD, q):
                    d.start()

            for d in gdescs(l, p):
                d.wait()

            for d in wdescs(l, p):
                d.start()
            return ()

        lax.fori_loop(0, L, body, (), unroll=False)
        for l in range(L - NBUF, L):
            for d in wdescs(l, l % NBUF):
                d.wait()

    return k(idxt, t0, t1, t2, t3)


def kernel(inputs, table0, table1, table2, table3):
    idxt = (inputs.astype(jnp.int32).transpose(2, 1, 0)
            .reshape(L, B // BLK, BLK, NUM_TABLES)
            .transpose(0, 1, 3, 2))
    out = _sc_gather_concat(idxt, table0, table1, table2, table3)
    return out.transpose(1, 0, 2)


# speedup vs baseline: 1.1639x; 1.0007x over previous
import functools

import jax
import jax.numpy as jnp
from jax import lax
from jax.experimental import pallas as pl
from jax.experimental.pallas import tpu as pltpu
from jax.experimental.pallas import tpu_sc as plsc

NUM_TABLES = 4
VOCAB = 100000
EMB = 32
B = 4096
L = 50

BLK = 128
NBUF = 4
GAHEAD = 2


def _sc_gather_concat(idxt, t0, t1, t2, t3):
    info = plsc.get_sparse_core_info()
    nc, ns = info.num_cores, info.num_subcores
    nw = nc * ns
    assert B == nw * BLK

    mesh = plsc.VectorSubcoreMesh(core_axis_name="c", subcore_axis_name="s")

    @functools.partial(
        pl.kernel,
        mesh=mesh,
        compiler_params=pltpu.CompilerParams(use_tc_tiling_on_sc=False,
                                             skip_device_barrier=True),
        out_type=jax.ShapeDtypeStruct((L, B, NUM_TABLES * EMB), jnp.float32),
        scratch_types=[
            pltpu.VMEM((L, NUM_TABLES, BLK), jnp.int32),
            pltpu.VMEM((NBUF, NUM_TABLES, BLK, EMB), jnp.float32),
            pltpu.SemaphoreType.DMA((NBUF,)),
            pltpu.SemaphoreType.DMA((NBUF,)),
        ],
    )
    def k(idx_hbm, tb0, tb1, tb2, tb3, out_hbm, idx_v, tmp_v, gsem, wsem):
        tables = (tb0, tb1, tb2, tb3)
        wid = lax.axis_index("s") * nc + lax.axis_index("c")
        b0 = wid * BLK

        pltpu.sync_copy(idx_hbm.at[:, wid], idx_v)

        def gdescs(l, p):
            return [pltpu.make_async_copy(
                tables[t].at[idx_v.at[l, t]],
                tmp_v.at[p, t],
                gsem.at[p]) for t in range(NUM_TABLES)]

        def wdescs(l, p):
            return [pltpu.make_async_copy(
                tmp_v.at[p, t],
                out_hbm.at[l, pl.ds(b0, BLK), pl.ds(t * EMB, EMB)],
                wsem.at[p]) for t in range(NUM_TABLES)]

        for l in range(GAHEAD):
            for d in gdescs(l, l):
                d.start()

        def body(l, _):
            p = lax.rem(l, NBUF)

            @pl.when(l + GAHEAD < L)
            def _():
                q = lax.rem(l + GAHEAD, NBUF)

                @pl.when(l - (NBUF - GAHEAD) >= 0)
                def _():
                    for d in wdescs(l - (NBUF - GAHEAD), q):
                        d.wait()

                for d in gdescs(l + GAHEAD, q):
                    d.start()

            for d in gdescs(l, p):
                d.wait()

            for d in wdescs(l, p):
                d.start()
            return ()

        lax.fori_loop(0, L, body, (), unroll=False)
        for l in range(L - NBUF, L):
            for d in wdescs(l, l % NBUF):
                d.wait()

    return k(idxt, t0, t1, t2, t3)


def kernel(inputs, table0, table1, table2, table3):
    idxt = (inputs.astype(jnp.int32).transpose(2, 1, 0)
            .reshape(L, B // BLK, BLK, NUM_TABLES)
            .transpose(0, 1, 3, 2))
    out = _sc_gather_concat(idxt, table0, table1, table2, table3)
    return out.transpose(1, 0, 2)
